# baseline (speedup 1.0000x reference)
import functools

import jax
import jax.numpy as jnp
from jax import lax
from jax.experimental import pallas as pl
from jax.experimental.pallas import tpu as pltpu
from jax.experimental.pallas import tpu_sc as plsc

_N = 50000
_CHUNK = 128
_NCHUNK = 391
_NPAD = _NCHUNK * _CHUNK
_BLK = 1088
_GRID = _NPAD // _BLK
_EPS = 1e-5
_NC, _NS = 2, 16
_NW = _NC * _NS

_POS_311 = (4, 13, 22)
_POS_131 = (10, 13, 16)
_POS_113 = (12, 13, 14)
_POS_133 = tuple(9 + 3 * dy + dx for dy in range(3) for dx in range(3))
_POS_313 = tuple(9 * dz + 3 + dx for dz in range(3) for dx in range(3))
_POS_A = tuple(sorted(set(_POS_311) | set(_POS_131) | set(_POS_113)))
_POS_C = tuple(sorted(set(_POS_313) | set(_POS_133)))
_COLA = {p: i for i, p in enumerate(_POS_A)}
_COLC = {p: i for i, p in enumerate(_POS_C)}
_SLOT_133 = tuple(_COLC[p] for p in _POS_133)
_SLOT_313 = tuple(_COLC[p] for p in _POS_313)


def _make_gather(P, W):
    G = 128 // W
    NP = (P + G - 1) // G
    PP = NP * G
    NV = W // 16
    mesh = plsc.VectorSubcoreMesh(core_axis_name="c", subcore_axis_name="s")

    @functools.partial(
        pl.kernel, mesh=mesh,
        out_type=jax.ShapeDtypeStruct((NP, _NPAD, 128), jnp.float32),
        scratch_types=[
            pltpu.VMEM((PP, _CHUNK), jnp.int32),
            pltpu.VMEM((_CHUNK, 128), jnp.float32),
            pltpu.VMEM((_CHUNK, 128), jnp.float32),
            pltpu.VMEM((_CHUNK, 128), jnp.float32),
            pltpu.SemaphoreType.DMA,
        ],
    )
    def gather_kernel(table, idx, out, idx_v, buf0, buf1, packed, gsem):
        wid = lax.axis_index("s") * _NC + lax.axis_index("c")
        nck = (_NCHUNK - wid + _NW - 1) // _NW
        bufs = (buf0, buf1)

        def pack(src, q):
            def body(rr, carry):
                for u in range(8):
                    r = rr * 8 + u
                    for v in range(NV):
                        packed[r, pl.ds(q * W + v * 16, 16)] = (
                            src[r, pl.ds(v * 16, 16)])
                return carry
            lax.fori_loop(0, _CHUNK // 8, body, 0)

        def step(t, carry):
            c = wid + t * _NW
            base = c * _CHUNK
            pltpu.sync_copy(idx.at[c], idx_v)
            cp = pltpu.async_copy(table.at[idx_v.at[0]], buf0, gsem)
            for j in range(PP):
                cp.wait()
                if j + 1 < PP:
                    nxt = pltpu.async_copy(
                        table.at[idx_v.at[j + 1]], bufs[(j + 1) % 2], gsem)
                if G == 1:
                    pltpu.sync_copy(bufs[j % 2],
                                    out.at[j, pl.ds(base, _CHUNK), :])
                else:
                    q = j % G
                    pack(bufs[j % 2], q)
                    if q == G - 1:
                        pltpu.sync_copy(
                            packed, out.at[j // G, pl.ds(base, _CHUNK), :])
                if j + 1 < PP:
                    cp = nxt
            return carry

        lax.fori_loop(0, nck, step, 0)

    return gather_kernel


_gather_cache = {}


def _gather(P, W):
    if (P, W) not in _gather_cache:
        _gather_cache[(P, W)] = _make_gather(P, W)
    return _gather_cache[(P, W)]


def _prep_idx(nbr, pos, PP):
    P = len(pos)
    cols = jnp.take(nbr, jnp.array(pos, jnp.int32), axis=1)
    if PP > P:
        cols = jnp.concatenate(
            [cols, jnp.zeros((_N, PP - P), jnp.int32)], axis=1)
    cols = jnp.concatenate(
        [cols, jnp.zeros((_NPAD - _N, PP), jnp.int32)], axis=0)
    return cols.reshape(_NCHUNK, _CHUNK, PP).transpose(0, 2, 1)


def _row_mask(i):
    rows = lax.broadcasted_iota(jnp.int32, (_BLK, 1), 0) + i * _BLK
    return (rows < _N).astype(jnp.float32)


def _acc_stats(ss_ref, val, mask, i):
    vm = val * mask
    s1 = jnp.sum(vm, axis=0, keepdims=True)
    s2 = jnp.sum(vm * val, axis=0, keepdims=True)

    @pl.when(i == 0)
    def _():
        ss_ref[...] = jnp.zeros_like(ss_ref)

    ss_ref[...] += jnp.concatenate([s1, s2], axis=0)


def _bn_coeffs(ss, g, b):
    mean = ss[0:1, :] * (1.0 / _N)
    var = ss[1:2, :] * (1.0 / _N) - mean * mean
    a = g * lax.rsqrt(var + _EPS)
    return a, b - mean * a


def _t_body(g_ref, w_ref, t_ref, ss_ref):
    i = pl.program_id(0)
    t = jax.nn.sigmoid(jnp.dot(g_ref[0], w_ref[...],
                               preferred_element_type=jnp.float32))
    t_ref[...] = t
    _acc_stats(ss_ref, t, _row_mask(i), i)


def _h_body(ss_ref, gb_ref, t_ref, h_ref):
    ss = ss_ref[...]
    gb = gb_ref[...]
    t = t_ref[...]
    h = jnp.zeros((_BLK, 32), jnp.float32)
    for j in range(3):
        a, sh = _bn_coeffs(ss[:, 32 * j:32 * j + 32],
                           gb[2 * j:2 * j + 1, :], gb[2 * j + 1:2 * j + 2, :])
        h = h + t[:, 32 * j:32 * j + 32] * a + sh
    h_ref[...] = jnp.concatenate([h, jnp.zeros((_BLK, 96), jnp.float32)], axis=1)


def _u_body(g_ref, w_ref, u_ref, ss_ref):
    i = pl.program_id(0)
    z = jnp.zeros((_BLK, 128), jnp.float32)
    for q in range(4):
        z = z + jnp.dot(g_ref[q], w_ref[128 * q:128 * q + 128, :],
                        preferred_element_type=jnp.float32)
    u = jnp.where(z > 0, z, 0.01 * z)
    u_ref[...] = u
    _acc_stats(ss_ref, u, _row_mask(i), i)


def _v_body(ssu_ref, gb_ref, w1_ref, w2_ref, ws1_ref, ws2_ref, g_ref,
            v_ref, ssv_ref):
    i = pl.program_id(0)
    ss = ssu_ref[...]
    gb = gb_ref[...]
    a1, c1 = _bn_coeffs(ss[:, :64], gb[0:1, :], gb[1:2, :])
    a2, c2 = _bn_coeffs(ss[:, 64:], gb[2:3, :], gb[3:4, :])
    v1 = jnp.dot(c1, ws1_ref[...], preferred_element_type=jnp.float32)
    v2 = jnp.dot(c2, ws2_ref[...], preferred_element_type=jnp.float32)
    v1 = jnp.broadcast_to(v1, (_BLK, 64))
    v2 = jnp.broadcast_to(v2, (_BLK, 64))
    for t in range(9):
        v1 = v1 + jnp.dot(g_ref[_SLOT_133[t]][:, :64] * a1,
                          w1_ref[64 * t:64 * t + 64, :],
                          preferred_element_type=jnp.float32)
        v2 = v2 + jnp.dot(g_ref[_SLOT_313[t]][:, 64:] * a2,
                          w2_ref[64 * t:64 * t + 64, :],
                          preferred_element_type=jnp.float32)
    v = jnp.concatenate([v1, v2], axis=1)
    v_ref[...] = v
    _acc_stats(ssv_ref, v, _row_mask(i), i)


def _o_body(ssv_ref, gb_ref, v_ref, o_ref):
    ss = ssv_ref[...]
    gb = gb_ref[...]
    a1, c1 = _bn_coeffs(ss[:, :64], gb[0:1, :], gb[1:2, :])
    a2, c2 = _bn_coeffs(ss[:, 64:], gb[2:3, :], gb[3:4, :])
    v = v_ref[...]
    o_ref[...] = v[:, :64] * a1 + c1 + v[:, 64:] * a2 + c2


def _full(shape):
    return pl.BlockSpec(shape, lambda i: tuple(0 for _ in shape))


_SEQ = pltpu.CompilerParams(dimension_semantics=("arbitrary",))


def kernel(x, Wr1, Wr2, Wr3, gr1, br1, gr2, br2, gr3, br3, Wc1, Wc12, Wc2,
           Wc3, gc0, bc0, gc02, bc02, gc1, bc1, gc2, bc2, nbr):
    f32 = jnp.float32

    WbigA = jnp.zeros((128, 96), f32)
    for tap in range(3):
        for W, pos, c0 in ((Wr1, _POS_311, 0), (Wr2, _POS_131, 32),
                           (Wr3, _POS_113, 64)):
            r = 16 * _COLA[pos[tap]]
            WbigA = WbigA.at[r:r + 16, c0:c0 + 32].set(W[tap])

    WbigC = jnp.zeros((512, 128), f32)
    for tap in range(9):
        s = _SLOT_313[tap]
        r = 128 * (s // 4) + 32 * (s % 4)
        WbigC = WbigC.at[r:r + 32, 0:64].set(Wc1[tap])
        s = _SLOT_133[tap]
        r = 128 * (s // 4) + 32 * (s % 4)
        WbigC = WbigC.at[r:r + 32, 64:128].set(Wc2[tap])

    Wstk1 = Wc12.reshape(9 * 64, 64)
    Wstk2 = Wc3.reshape(9 * 64, 64)
    Wsum1 = jnp.sum(Wc12, axis=0)
    Wsum2 = jnp.sum(Wc3, axis=0)
    GBr = jnp.stack([gr1, br1, gr2, br2, gr3, br3])
    GBu = jnp.stack([gc0, bc0, gc1, bc1])
    GBo = jnp.stack([gc02, bc02, gc2, bc2])

    xp = jnp.pad(x, ((0, 0), (0, 128 - 16)))
    G_A = _gather(7, 16)(xp, _prep_idx(nbr, _POS_A, 8))
    T, ss_t = pl.pallas_call(
        _t_body, grid=(_GRID,),
        in_specs=[pl.BlockSpec((1, _BLK, 128), lambda i: (0, i, 0)),
                  _full((128, 96))],
        out_specs=[pl.BlockSpec((_BLK, 96), lambda i: (i, 0)), _full((2, 96))],
        out_shape=[jax.ShapeDtypeStruct((_NPAD, 96), f32),
                   jax.ShapeDtypeStruct((2, 96), f32)],
        compiler_params=_SEQ,
    )(G_A, WbigA)

    h = pl.pallas_call(
        _h_body, grid=(_GRID,),
        in_specs=[_full((2, 96)), _full((6, 32)),
                  pl.BlockSpec((_BLK, 96), lambda i: (i, 0))],
        out_specs=pl.BlockSpec((_BLK, 128), lambda i: (i, 0)),
        out_shape=jax.ShapeDtypeStruct((_NPAD, 128), f32),
        compiler_params=_SEQ,
    )(ss_t, GBr, T)

    G_C = _gather(15, 32)(h, _prep_idx(nbr, _POS_C, 16))
    U, ss_u = pl.pallas_call(
        _u_body, grid=(_GRID,),
        in_specs=[pl.BlockSpec((4, _BLK, 128), lambda i: (0, i, 0)),
                  _full((512, 128))],
        out_specs=[pl.BlockSpec((_BLK, 128), lambda i: (i, 0)),
                   _full((2, 128))],
        out_shape=[jax.ShapeDtypeStruct((_NPAD, 128), f32),
                   jax.ShapeDtypeStruct((2, 128), f32)],
        compiler_params=_SEQ,
    )(G_C, WbigC)

    G_D = _gather(15, 128)(U, _prep_idx(nbr, _POS_C, 15))
    V, ss_v = pl.pallas_call(
        _v_body, grid=(_GRID,),
        in_specs=[_full((2, 128)), _full((4, 64)), _full((576, 64)),
                  _full((576, 64)), _full((64, 64)), _full((64, 64)),
                  pl.BlockSpec((15, _BLK, 128), lambda i: (0, i, 0))],
        out_specs=[pl.BlockSpec((_BLK, 128), lambda i: (i, 0)),
                   _full((2, 128))],
        out_shape=[jax.ShapeDtypeStruct((_NPAD, 128), f32),
                   jax.ShapeDtypeStruct((2, 128), f32)],
        compiler_params=_SEQ,
    )(ss_u, GBu, Wstk1, Wstk2, Wsum1, Wsum2, G_D)

    out = pl.pallas_call(
        _o_body, grid=(_GRID,),
        in_specs=[_full((2, 128)), _full((4, 64)),
                  pl.BlockSpec((_BLK, 128), lambda i: (i, 0))],
        out_specs=pl.BlockSpec((_BLK, 64), lambda i: (i, 0)),
        out_shape=jax.ShapeDtypeStruct((_N, 64), f32),
        compiler_params=_SEQ,
    )(ss_v, GBo, V)
    return out


# speedup vs baseline: 3.4118x; 3.4118x over previous
import functools

import jax
import jax.numpy as jnp
from jax import lax
from jax.experimental import pallas as pl
from jax.experimental.pallas import tpu as pltpu
from jax.experimental.pallas import tpu_sc as plsc

_N = 50000
_CHUNK = 128
_NCHUNK = 391
_NPAD = _NCHUNK * _CHUNK
_BLK = 1088
_GRID = _NPAD // _BLK
_EPS = 1e-5
_NC, _NS = 2, 16
_NW = _NC * _NS

_POS_311 = (4, 13, 22)
_POS_131 = (10, 13, 16)
_POS_113 = (12, 13, 14)
_POS_133 = tuple(9 + 3 * dy + dx for dy in range(3) for dx in range(3))
_POS_313 = tuple(9 * dz + 3 + dx for dz in range(3) for dx in range(3))
_POS_A = tuple(sorted(set(_POS_311) | set(_POS_131) | set(_POS_113)))
_POS_C = tuple(sorted(set(_POS_313) | set(_POS_133)))
_COLC = {p: i for i, p in enumerate(_POS_C)}
_SLOT_133 = tuple(_COLC[p] for p in _POS_133)
_SLOT_313 = tuple(_COLC[p] for p in _POS_313)


def _make_gather(P):
    mesh = plsc.VectorSubcoreMesh(core_axis_name="c", subcore_axis_name="s")

    @functools.partial(
        pl.kernel, mesh=mesh,
        out_type=jax.ShapeDtypeStruct((P, _NPAD, 128), jnp.float32),
        scratch_types=[
            pltpu.VMEM((P, _CHUNK), jnp.int32),
            pltpu.VMEM((_CHUNK, 128), jnp.float32),
            pltpu.VMEM((_CHUNK, 128), jnp.float32),
            pltpu.SemaphoreType.DMA,
        ],
    )
    def gather_kernel(table, idx, out, idx_v, buf0, buf1, gsem):
        wid = lax.axis_index("s") * _NC + lax.axis_index("c")
        nck = (_NCHUNK - wid + _NW - 1) // _NW
        bufs = (buf0, buf1)

        def step(t, carry):
            c = wid + t * _NW
            base = c * _CHUNK
            pltpu.sync_copy(idx.at[c], idx_v)
            cp = pltpu.async_copy(table.at[idx_v.at[0]], buf0, gsem)
            for j in range(P):
                cp.wait()
                if j + 1 < P:
                    nxt = pltpu.async_copy(
                        table.at[idx_v.at[j + 1]], bufs[(j + 1) % 2], gsem)
                pltpu.sync_copy(bufs[j % 2], out.at[j, pl.ds(base, _CHUNK), :])
                if j + 1 < P:
                    cp = nxt
            return carry

        lax.fori_loop(0, nck, step, 0)

    return gather_kernel


_gather_cache = {}


def _gather(P):
    if P not in _gather_cache:
        _gather_cache[P] = _make_gather(P)
    return _gather_cache[P]


def _prep_idx(nbr, pos):
    P = len(pos)
    cols = jnp.take(nbr, jnp.array(pos, jnp.int32), axis=1)
    cols = jnp.concatenate(
        [cols, jnp.zeros((_NPAD - _N, P), jnp.int32)], axis=0)
    return cols.reshape(_NCHUNK, _CHUNK, P).transpose(0, 2, 1)


def _row_mask(i):
    rows = lax.broadcasted_iota(jnp.int32, (_BLK, 1), 0) + i * _BLK
    return (rows < _N).astype(jnp.float32)


def _acc_stats(ss_ref, val, mask, i):
    vm = val * mask
    s1 = jnp.sum(vm, axis=0, keepdims=True)
    s2 = jnp.sum(vm * val, axis=0, keepdims=True)

    @pl.when(i == 0)
    def _():
        ss_ref[...] = jnp.zeros_like(ss_ref)

    ss_ref[...] += jnp.concatenate([s1, s2], axis=0)


def _bn_coeffs(ss, g, b):
    mean = ss[0:1, :] * (1.0 / _N)
    var = ss[1:2, :] * (1.0 / _N) - mean * mean
    a = g * lax.rsqrt(var + _EPS)
    return a, b - mean * a


def _t_body(g_ref, w_ref, t_ref, ss_ref):
    i = pl.program_id(0)
    z = jnp.zeros((_BLK, 96), jnp.float32)
    for p in range(7):
        z = z + jnp.dot(g_ref[p][:, :16], w_ref[16 * p:16 * p + 16, :],
                        preferred_element_type=jnp.float32)
    t = jax.nn.sigmoid(z)
    t_ref[...] = t
    _acc_stats(ss_ref, t, _row_mask(i), i)


def _h_body(ss_ref, gb_ref, t_ref, h_ref):
    ss = ss_ref[...]
    gb = gb_ref[...]
    t = t_ref[...]
    h = jnp.zeros((_BLK, 32), jnp.float32)
    for j in range(3):
        a, sh = _bn_coeffs(ss[:, 32 * j:32 * j + 32],
                           gb[2 * j:2 * j + 1, :], gb[2 * j + 1:2 * j + 2, :])
        h = h + t[:, 32 * j:32 * j + 32] * a + sh
    h_ref[...] = jnp.concatenate([h, jnp.zeros((_BLK, 96), jnp.float32)], axis=1)


def _u_body(g_ref, w_ref, u_ref, ss_ref):
    i = pl.program_id(0)
    z = jnp.zeros((_BLK, 128), jnp.float32)
    for s in range(15):
        z = z + jnp.dot(g_ref[s][:, :32], w_ref[32 * s:32 * s + 32, :],
                        preferred_element_type=jnp.float32)
    u = jnp.where(z > 0, z, 0.01 * z)
    u_ref[...] = u
    _acc_stats(ss_ref, u, _row_mask(i), i)


def _v_body(ssu_ref, gb_ref, w1_ref, w2_ref, ws1_ref, ws2_ref, g_ref,
            v_ref, ssv_ref):
    i = pl.program_id(0)
    ss = ssu_ref[...]
    gb = gb_ref[...]
    a1, c1 = _bn_coeffs(ss[:, :64], gb[0:1, :], gb[1:2, :])
    a2, c2 = _bn_coeffs(ss[:, 64:], gb[2:3, :], gb[3:4, :])
    v1 = jnp.dot(c1, ws1_ref[...], preferred_element_type=jnp.float32)
    v2 = jnp.dot(c2, ws2_ref[...], preferred_element_type=jnp.float32)
    v1 = jnp.broadcast_to(v1, (_BLK, 64))
    v2 = jnp.broadcast_to(v2, (_BLK, 64))
    for t in range(9):
        v1 = v1 + jnp.dot(g_ref[_SLOT_133[t]][:, :64] * a1,
                          w1_ref[64 * t:64 * t + 64, :],
                          preferred_element_type=jnp.float32)
        v2 = v2 + jnp.dot(g_ref[_SLOT_313[t]][:, 64:] * a2,
                          w2_ref[64 * t:64 * t + 64, :],
                          preferred_element_type=jnp.float32)
    v = jnp.concatenate([v1, v2], axis=1)
    v_ref[...] = v
    _acc_stats(ssv_ref, v, _row_mask(i), i)


def _o_body(ssv_ref, gb_ref, v_ref, o_ref):
    ss = ssv_ref[...]
    gb = gb_ref[...]
    a1, c1 = _bn_coeffs(ss[:, :64], gb[0:1, :], gb[1:2, :])
    a2, c2 = _bn_coeffs(ss[:, 64:], gb[2:3, :], gb[3:4, :])
    v = v_ref[...]
    o_ref[...] = v[:, :64] * a1 + c1 + v[:, 64:] * a2 + c2


def _full(shape):
    return pl.BlockSpec(shape, lambda i: tuple(0 for _ in shape))


_SEQ = pltpu.CompilerParams(dimension_semantics=("arbitrary",))


def kernel(x, Wr1, Wr2, Wr3, gr1, br1, gr2, br2, gr3, br3, Wc1, Wc12, Wc2,
           Wc3, gc0, bc0, gc02, bc02, gc1, bc1, gc2, bc2, nbr):
    f32 = jnp.float32

    colA = {p: k for k, p in enumerate(_POS_A)}
    WbigA = jnp.zeros((7 * 16, 96), f32)
    for tap in range(3):
        for W, pos, c0 in ((Wr1, _POS_311, 0), (Wr2, _POS_131, 32),
                           (Wr3, _POS_113, 64)):
            r = 16 * colA[pos[tap]]
            WbigA = WbigA.at[r:r + 16, c0:c0 + 32].set(W[tap])

    WbigC = jnp.zeros((15 * 32, 128), f32)
    for tap in range(9):
        r = 32 * _COLC[_POS_313[tap]]
        WbigC = WbigC.at[r:r + 32, 0:64].set(Wc1[tap])
        r = 32 * _COLC[_POS_133[tap]]
        WbigC = WbigC.at[r:r + 32, 64:128].set(Wc2[tap])

    Wstk1 = Wc12.reshape(9 * 64, 64)
    Wstk2 = Wc3.reshape(9 * 64, 64)
    Wsum1 = jnp.sum(Wc12, axis=0)
    Wsum2 = jnp.sum(Wc3, axis=0)
    GBr = jnp.stack([gr1, br1, gr2, br2, gr3, br3])
    GBu = jnp.stack([gc0, bc0, gc1, bc1])
    GBo = jnp.stack([gc02, bc02, gc2, bc2])

    xp = jnp.pad(x, ((0, 0), (0, 128 - 16)))
    G_A = _gather(7)(xp, _prep_idx(nbr, _POS_A))
    T, ss_t = pl.pallas_call(
        _t_body, grid=(_GRID,),
        in_specs=[pl.BlockSpec((7, _BLK, 128), lambda i: (0, i, 0)),
                  _full((112, 96))],
        out_specs=[pl.BlockSpec((_BLK, 96), lambda i: (i, 0)), _full((2, 96))],
        out_shape=[jax.ShapeDtypeStruct((_NPAD, 96), f32),
                   jax.ShapeDtypeStruct((2, 96), f32)],
        compiler_params=_SEQ,
    )(G_A, WbigA)

    h = pl.pallas_call(
        _h_body, grid=(_GRID,),
        in_specs=[_full((2, 96)), _full((6, 32)),
                  pl.BlockSpec((_BLK, 96), lambda i: (i, 0))],
        out_specs=pl.BlockSpec((_BLK, 128), lambda i: (i, 0)),
        out_shape=jax.ShapeDtypeStruct((_NPAD, 128), f32),
        compiler_params=_SEQ,
    )(ss_t, GBr, T)

    G_C = _gather(15)(h, _prep_idx(nbr, _POS_C))
    U, ss_u = pl.pallas_call(
        _u_body, grid=(_GRID,),
        in_specs=[pl.BlockSpec((15, _BLK, 128), lambda i: (0, i, 0)),
                  _full((480, 128))],
        out_specs=[pl.BlockSpec((_BLK, 128), lambda i: (i, 0)),
                   _full((2, 128))],
        out_shape=[jax.ShapeDtypeStruct((_NPAD, 128), f32),
                   jax.ShapeDtypeStruct((2, 128), f32)],
        compiler_params=_SEQ,
    )(G_C, WbigC)

    G_D = _gather(15)(U, _prep_idx(nbr, _POS_C))
    V, ss_v = pl.pallas_call(
        _v_body, grid=(_GRID,),
        in_specs=[_full((2, 128)), _full((4, 64)), _full((576, 64)),
                  _full((576, 64)), _full((64, 64)), _full((64, 64)),
                  pl.BlockSpec((15, _BLK, 128), lambda i: (0, i, 0))],
        out_specs=[pl.BlockSpec((_BLK, 128), lambda i: (i, 0)),
                   _full((2, 128))],
        out_shape=[jax.ShapeDtypeStruct((_NPAD, 128), f32),
                   jax.ShapeDtypeStruct((2, 128), f32)],
        compiler_params=_SEQ,
    )(ss_u, GBu, Wstk1, Wstk2, Wsum1, Wsum2, G_D)

    out = pl.pallas_call(
        _o_body, grid=(_GRID,),
        in_specs=[_full((2, 128)), _full((4, 64)),
                  pl.BlockSpec((_BLK, 128), lambda i: (i, 0))],
        out_specs=pl.BlockSpec((_BLK, 64), lambda i: (i, 0)),
        out_shape=jax.ShapeDtypeStruct((_N, 64), f32),
        compiler_params=_SEQ,
    )(ss_v, GBo, V)
    return out
